# baseline (speedup 1.0000x reference)
import jax
import jax.numpy as jnp
from jax.experimental import pallas as pl
from jax.experimental.pallas import tpu as pltpu

_BB = 2048
_C = 4
_NACC = 8


def _encode(x2, w1e, b1e, w2e, b2e, w3e, b3, n_set, d_in):
    f32 = jnp.float32
    bf = jnp.bfloat16
    k1 = _C * d_in
    nch = n_set // _C
    sacc = [None] * _NACC
    macc = [None] * _NACC
    for c in range(nch):
        xc = x2[:, c * k1:(c + 1) * k1]
        h = jnp.dot(xc, w1e, preferred_element_type=f32).astype(bf)
        h = jnp.maximum(h + b1e, 0.0)
        h = jnp.dot(h, w2e, preferred_element_type=f32).astype(bf)
        h = jnp.maximum(h + b2e, 0.0)
        e = jnp.dot(h, w3e, preferred_element_type=f32)
        a = c % _NACC
        sacc[a] = e if sacc[a] is None else sacc[a] + e
        macc[a] = e if macc[a] is None else jnp.maximum(macc[a], e)
    st = sacc[0]
    mt = macc[0]
    for a in range(1, _NACC):
        st = st + sacc[a]
        mt = jnp.maximum(mt, macc[a])
    s = st[:, :32]
    m = mt[:, :32]
    for j in range(1, _C):
        s = s + st[:, j * 32:(j + 1) * 32]
        m = jnp.maximum(m, mt[:, j * 32:(j + 1) * 32])
    return (s * (1.0 / n_set) + m) * 0.5 + b3


def _fused_kernel(t0_ref, rx_ref, tx_ref,
                  rw1e_ref, rb1e_ref, rw2e_ref, rb2e_ref, rw3e_ref, rb3_ref,
                  tw1e_ref, tb1e_ref, tw2e_ref, tb2e_ref, tw3e_ref, tb3_ref,
                  mw1a_ref, mw1b_ref, mw1c_ref, mb1_ref,
                  mw2_ref, mb2_ref, mw3_ref, mb3_ref, mw4_ref, mb4_ref,
                  out_ref):
    f32 = jnp.float32
    r_emb = _encode(rx_ref[...], rw1e_ref[...], rb1e_ref[...], rw2e_ref[...],
                    rb2e_ref[...], rw3e_ref[...], rb3_ref[...], 64, 6)
    t_emb = _encode(tx_ref[...], tw1e_ref[...], tb1e_ref[...], tw2e_ref[...],
                    tb2e_ref[...], tw3e_ref[...], tb3_ref[...], 128, 7)

    h = (jnp.dot(t0_ref[...], mw1a_ref[...], preferred_element_type=f32)
         + jnp.dot(r_emb, mw1b_ref[...], preferred_element_type=f32)
         + jnp.dot(t_emb, mw1c_ref[...], preferred_element_type=f32)
         + mb1_ref[...])
    h = jnp.maximum(h, 0.0)
    h = jnp.maximum(jnp.dot(h, mw2_ref[...], preferred_element_type=f32)
                    + mb2_ref[...], 0.0)
    h = jnp.maximum(jnp.dot(h, mw3_ref[...], preferred_element_type=f32)
                    + mb3_ref[...], 0.0)
    out_ref[...] = jnp.dot(h, mw4_ref[...], preferred_element_type=f32) + mb4_ref[...]


def _blockdiag(w, c):
    k, n = w.shape
    eye = jnp.eye(c, dtype=w.dtype)
    return (eye[:, None, :, None] * w[None, :, None, :]).reshape(c * k, c * n)


def kernel(tier0_features, robot_features, track_features,
           rw1, rb1, rw2, rb2, rw3, rb3,
           tw1, tb1, tw2, tb2, tw3, tb3,
           mw1, mb1, mw2, mb2, mw3, mb3, mw4, mb4):
    B = tier0_features.shape[0]
    grid = (B // _BB,)
    bf = jnp.bfloat16

    x2r = robot_features.astype(bf).reshape(B, 64 * 6)
    x2t = track_features.astype(bf).reshape(B, 128 * 7)

    rw1e, rw2e, rw3e = (_blockdiag(rw1, _C).astype(bf),
                        _blockdiag(rw2, _C).astype(bf),
                        _blockdiag(rw3, _C).astype(bf))
    tw1e, tw2e, tw3e = (_blockdiag(tw1, _C).astype(bf),
                        _blockdiag(tw2, _C).astype(bf),
                        _blockdiag(tw3, _C).astype(bf))
    rb1e, rb2e = jnp.tile(rb1, _C)[None].astype(bf), jnp.tile(rb2, _C)[None].astype(bf)
    tb1e, tb2e = jnp.tile(tb1, _C)[None].astype(bf), jnp.tile(tb2, _C)[None].astype(bf)
    rb3_, tb3_ = rb3[None], tb3[None]
    mb1_, mb2_, mb3_, mb4_ = mb1[None], mb2[None], mb3[None], mb4[None]
    mw1a, mw1b, mw1c = mw1[:44], mw1[44:76], mw1[76:108]

    def rows(i):
        return (i, 0)

    def full2(i):
        return (0, 0)

    w2 = lambda shape: pl.BlockSpec(shape, full2)

    out = pl.pallas_call(
        _fused_kernel,
        grid=grid,
        in_specs=[
            pl.BlockSpec((_BB, 44), rows),
            pl.BlockSpec((_BB, 64 * 6), rows),
            pl.BlockSpec((_BB, 128 * 7), rows),
            w2((_C * 6, _C * 64)), w2((1, _C * 64)),
            w2((_C * 64, _C * 64)), w2((1, _C * 64)),
            w2((_C * 64, _C * 32)), w2((1, 32)),
            w2((_C * 7, _C * 64)), w2((1, _C * 64)),
            w2((_C * 64, _C * 64)), w2((1, _C * 64)),
            w2((_C * 64, _C * 32)), w2((1, 32)),
            w2((44, 128)), w2((32, 128)), w2((32, 128)), w2((1, 128)),
            w2((128, 128)), w2((1, 128)), w2((128, 64)), w2((1, 64)),
            w2((64, 1)), w2((1, 1)),
        ],
        out_specs=pl.BlockSpec((_BB, 1), rows),
        out_shape=jax.ShapeDtypeStruct((B, 1), jnp.float32),
        compiler_params=pltpu.CompilerParams(
            dimension_semantics=("arbitrary",),
            vmem_limit_bytes=100 * 1024 * 1024,
        ),
    )(tier0_features, x2r, x2t,
      rw1e, rb1e, rw2e, rb2e, rw3e, rb3_,
      tw1e, tb1e, tw2e, tb2e, tw3e, tb3_,
      mw1a, mw1b, mw1c, mb1_,
      mw2, mb2_, mw3, mb3_, mw4, mb4_)
    return out[:, 0]


# speedup vs baseline: 1.1754x; 1.1754x over previous
import jax
import jax.numpy as jnp
from jax.experimental import pallas as pl
from jax.experimental.pallas import tpu as pltpu

_BB = 1024
_C = 4
_NACC = 8


def _encode(x2, w1e, b1e, w2e, b2e, w3e, b3, n_set, d_in):
    f32 = jnp.float32
    bf = jnp.bfloat16
    k1 = _C * d_in
    nch = n_set // _C
    sacc = [None] * _NACC
    macc = [None] * _NACC
    for c in range(nch):
        xc = x2[:, c * k1:(c + 1) * k1]
        h = jnp.dot(xc, w1e, preferred_element_type=f32).astype(bf)
        h = jnp.maximum(h + b1e, 0.0)
        h = jnp.dot(h, w2e, preferred_element_type=f32).astype(bf)
        h = jnp.maximum(h + b2e, 0.0)
        e = jnp.dot(h, w3e, preferred_element_type=f32)
        a = c % _NACC
        sacc[a] = e if sacc[a] is None else sacc[a] + e
        macc[a] = e if macc[a] is None else jnp.maximum(macc[a], e)
    st = sacc[0]
    mt = macc[0]
    for a in range(1, _NACC):
        st = st + sacc[a]
        mt = jnp.maximum(mt, macc[a])
    s = st[:, :32]
    m = mt[:, :32]
    for j in range(1, _C):
        s = s + st[:, j * 32:(j + 1) * 32]
        m = jnp.maximum(m, mt[:, j * 32:(j + 1) * 32])
    return (s * (1.0 / n_set) + m) * 0.5 + b3


def _fused_kernel(t0_ref, rx_ref, tx_ref,
                  rw1e_ref, rb1e_ref, rw2e_ref, rb2e_ref, rw3e_ref, rb3_ref,
                  tw1e_ref, tb1e_ref, tw2e_ref, tb2e_ref, tw3e_ref, tb3_ref,
                  mw1a_ref, mw1b_ref, mw1c_ref, mb1_ref,
                  mw2_ref, mb2_ref, mw3_ref, mb3_ref, mw4_ref, mb4_ref,
                  out_ref):
    f32 = jnp.float32
    r_emb = _encode(rx_ref[...], rw1e_ref[...], rb1e_ref[...], rw2e_ref[...],
                    rb2e_ref[...], rw3e_ref[...], rb3_ref[...], 64, 6)
    t_emb = _encode(tx_ref[...], tw1e_ref[...], tb1e_ref[...], tw2e_ref[...],
                    tb2e_ref[...], tw3e_ref[...], tb3_ref[...], 128, 7)

    h = (jnp.dot(t0_ref[...], mw1a_ref[...], preferred_element_type=f32)
         + jnp.dot(r_emb, mw1b_ref[...], preferred_element_type=f32)
         + jnp.dot(t_emb, mw1c_ref[...], preferred_element_type=f32)
         + mb1_ref[...])
    h = jnp.maximum(h, 0.0)
    h = jnp.maximum(jnp.dot(h, mw2_ref[...], preferred_element_type=f32)
                    + mb2_ref[...], 0.0)
    h = jnp.maximum(jnp.dot(h, mw3_ref[...], preferred_element_type=f32)
                    + mb3_ref[...], 0.0)
    out_ref[...] = jnp.dot(h, mw4_ref[...], preferred_element_type=f32) + mb4_ref[...]


def _blockdiag(w, c):
    k, n = w.shape
    eye = jnp.eye(c, dtype=w.dtype)
    return (eye[:, None, :, None] * w[None, :, None, :]).reshape(c * k, c * n)


def kernel(tier0_features, robot_features, track_features,
           rw1, rb1, rw2, rb2, rw3, rb3,
           tw1, tb1, tw2, tb2, tw3, tb3,
           mw1, mb1, mw2, mb2, mw3, mb3, mw4, mb4):
    B = tier0_features.shape[0]
    grid = (B // _BB,)
    bf = jnp.bfloat16

    x2r = robot_features.astype(bf).reshape(B, 64 * 6)
    x2t = track_features.astype(bf).reshape(B, 128 * 7)

    rw1e, rw2e, rw3e = (_blockdiag(rw1, _C).astype(bf),
                        _blockdiag(rw2, _C).astype(bf),
                        _blockdiag(rw3, _C).astype(bf))
    tw1e, tw2e, tw3e = (_blockdiag(tw1, _C).astype(bf),
                        _blockdiag(tw2, _C).astype(bf),
                        _blockdiag(tw3, _C).astype(bf))
    rb1e, rb2e = jnp.tile(rb1, _C)[None].astype(bf), jnp.tile(rb2, _C)[None].astype(bf)
    tb1e, tb2e = jnp.tile(tb1, _C)[None].astype(bf), jnp.tile(tb2, _C)[None].astype(bf)
    rb3_, tb3_ = rb3[None], tb3[None]
    mb1_, mb2_, mb3_, mb4_ = mb1[None], mb2[None], mb3[None], mb4[None]
    mw1a, mw1b, mw1c = mw1[:44], mw1[44:76], mw1[76:108]

    def rows(i):
        return (i, 0)

    def full2(i):
        return (0, 0)

    w2 = lambda shape: pl.BlockSpec(shape, full2)

    out = pl.pallas_call(
        _fused_kernel,
        grid=grid,
        in_specs=[
            pl.BlockSpec((_BB, 44), rows),
            pl.BlockSpec((_BB, 64 * 6), rows),
            pl.BlockSpec((_BB, 128 * 7), rows),
            w2((_C * 6, _C * 64)), w2((1, _C * 64)),
            w2((_C * 64, _C * 64)), w2((1, _C * 64)),
            w2((_C * 64, _C * 32)), w2((1, 32)),
            w2((_C * 7, _C * 64)), w2((1, _C * 64)),
            w2((_C * 64, _C * 64)), w2((1, _C * 64)),
            w2((_C * 64, _C * 32)), w2((1, 32)),
            w2((44, 128)), w2((32, 128)), w2((32, 128)), w2((1, 128)),
            w2((128, 128)), w2((1, 128)), w2((128, 64)), w2((1, 64)),
            w2((64, 1)), w2((1, 1)),
        ],
        out_specs=pl.BlockSpec((_BB, 1), rows),
        out_shape=jax.ShapeDtypeStruct((B, 1), jnp.float32),
        compiler_params=pltpu.CompilerParams(
            dimension_semantics=("arbitrary",),
            vmem_limit_bytes=100 * 1024 * 1024,
        ),
    )(tier0_features, x2r, x2t,
      rw1e, rb1e, rw2e, rb2e, rw3e, rb3_,
      tw1e, tb1e, tw2e, tb2e, tw3e, tb3_,
      mw1a, mw1b, mw1c, mb1_,
      mw2, mb2_, mw3, mb3_, mw4, mb4_)
    return out[:, 0]
